# Initial kernel scaffold; baseline (speedup 1.0000x reference)
import jax
import jax.numpy as jnp
from jax.experimental import pallas as pl


def kernel(roi_feats, W1, att_src1, att_dst1, b1, W2, att_src2, att_dst2, b2, fcW, fcb):
    raise NotImplementedError("write your pallas kernel here")


# speedup vs baseline: 1546.0025x; 1546.0025x over previous
import jax
import jax.numpy as jnp
from jax.experimental import pallas as pl

N = 384
HID = 768
GH = 128
HEADS = 4

_PREC = jax.lax.Precision.HIGHEST


def _lrelu(x):
    return jnp.where(x >= 0, x, 0.2 * x)


def _elu(x):
    return jnp.where(x > 0, x, jnp.exp(jnp.minimum(x, 0.0)) - 1.0)


def _attend(h, a_src, a_dst):
    s = jnp.sum(h * a_src, axis=1, keepdims=True)
    d = jnp.sum(h * a_dst, axis=1, keepdims=True)
    logit = _lrelu(d + s.T)
    m = jnp.max(logit, axis=1, keepdims=True)
    e = jnp.exp(logit - m)
    denom = jnp.sum(e, axis=1, keepdims=True) + 1e-16
    coef = e / denom
    return jnp.dot(coef, h, precision=_PREC)


def _gat_kernel(x_ref, w1_ref, as1_ref, ad1_ref, b1_ref, w2_ref, as2_ref,
                ad2_ref, b2_ref, fcw_ref, fcb_ref, out_ref):
    x = x_ref[...]
    h1 = jnp.dot(x, w1_ref[...], precision=_PREC)

    outs = []
    for k in range(HEADS):
        hk = h1[:, k * GH:(k + 1) * GH]
        outs.append(_attend(hk, as1_ref[k:k + 1, :], ad1_ref[k:k + 1, :]))
    x1 = _elu(jnp.concatenate(outs, axis=1) + b1_ref[...])

    h2 = jnp.dot(x1, w2_ref[...], precision=_PREC)
    o2 = _elu(_attend(h2, as2_ref[...], ad2_ref[...]) + b2_ref[...])

    g = jnp.mean(o2, axis=0, keepdims=True)
    out_ref[...] = jnp.dot(g, fcw_ref[...].T, precision=_PREC) + fcb_ref[...]


def kernel(roi_feats, W1, att_src1, att_dst1, b1, W2, att_src2, att_dst2, b2,
           fcW, fcb):
    out = pl.pallas_call(
        _gat_kernel,
        out_shape=jax.ShapeDtypeStruct((1, HID), jnp.float32),
    )(roi_feats, W1, att_src1, att_dst1, b1.reshape(1, -1), W2, att_src2,
      att_dst2, b2.reshape(1, -1), fcW, fcb.reshape(1, -1))
    return out.reshape(HID)


# speedup vs baseline: 2409.0239x; 1.5582x over previous
import jax
import jax.numpy as jnp
from jax.experimental import pallas as pl

N = 384
HID = 768
GH = 128
HEADS = 4

_PREC = jax.lax.Precision.DEFAULT


def _lrelu(x):
    return jnp.where(x >= 0, x, 0.2 * x)


def _elu(x):
    return jnp.where(x > 0, x, jnp.exp(jnp.minimum(x, 0.0)) - 1.0)


def _attend(h, a_src, a_dst):
    s = jnp.sum(h * a_src, axis=1, keepdims=True)
    d = jnp.sum(h * a_dst, axis=1, keepdims=True)
    logit = _lrelu(d + s.T)
    m = jnp.max(logit, axis=1, keepdims=True)
    e = jnp.exp(logit - m)
    denom = jnp.sum(e, axis=1, keepdims=True) + 1e-16
    coef = e / denom
    return jnp.dot(coef, h, precision=_PREC)


def _gat_kernel(x_ref, w1_ref, as1_ref, ad1_ref, b1_ref, w2_ref, as2_ref,
                ad2_ref, b2_ref, fcw_ref, fcb_ref, out_ref):
    x = x_ref[...]
    h1 = jnp.dot(x, w1_ref[...], precision=_PREC)

    outs = []
    for k in range(HEADS):
        hk = h1[:, k * GH:(k + 1) * GH]
        outs.append(_attend(hk, as1_ref[k:k + 1, :], ad1_ref[k:k + 1, :]))
    x1 = _elu(jnp.concatenate(outs, axis=1) + b1_ref[...])

    h2 = jnp.dot(x1, w2_ref[...], precision=_PREC)
    o2 = _elu(_attend(h2, as2_ref[...], ad2_ref[...]) + b2_ref[...])

    g = jnp.mean(o2, axis=0, keepdims=True)
    out_ref[...] = jnp.dot(g, fcw_ref[...].T, precision=_PREC) + fcb_ref[...]


def kernel(roi_feats, W1, att_src1, att_dst1, b1, W2, att_src2, att_dst2, b2,
           fcW, fcb):
    out = pl.pallas_call(
        _gat_kernel,
        out_shape=jax.ShapeDtypeStruct((1, HID), jnp.float32),
    )(roi_feats, W1, att_src1, att_dst1, b1.reshape(1, -1), W2, att_src2,
      att_dst2, b2.reshape(1, -1), fcW, fcb.reshape(1, -1))
    return out.reshape(HID)


# speedup vs baseline: 3271.8145x; 1.3581x over previous
import jax
import jax.numpy as jnp
from jax.experimental import pallas as pl

N = 384
HID = 768
GH = 128
HEADS = 4

_DN = (((1,), (1,)), ((), ()))


def _lrelu(x):
    return jnp.maximum(x, 0.2 * x)


def _elu(x):
    return jnp.maximum(x, 0.0) + jnp.exp(jnp.minimum(x, 0.0)) - 1.0


def _attend(h, a_src, a_dst, ones_col):
    s = jax.lax.dot_general(a_src, h, _DN)
    d = jax.lax.dot_general(h, a_dst, _DN)
    logit = _lrelu(d + s)
    m = jnp.max(logit, axis=1, keepdims=True)
    e = jnp.exp(logit - m)
    denom = jnp.dot(e, ones_col)
    agg = jnp.dot(e, h)
    return agg * (1.0 / (denom + 1e-16))


def _gat_kernel(x_ref, w1_ref, as1_ref, ad1_ref, b1_ref, w2_ref, as2_ref,
                ad2_ref, b2_ref, fcw_ref, fcb_ref, out_ref):
    ones_col = jnp.ones((N, 1), jnp.float32)
    h1 = jnp.dot(x_ref[...], w1_ref[...])

    outs = []
    for k in range(HEADS):
        hk = h1[:, k * GH:(k + 1) * GH]
        outs.append(_attend(hk, as1_ref[k:k + 1, :], ad1_ref[k:k + 1, :],
                            ones_col))
    x1 = _elu(jnp.concatenate(outs, axis=1) + b1_ref[...].reshape(1, -1))

    h2 = jnp.dot(x1, w2_ref[...])
    o2 = _elu(_attend(h2, as2_ref[...], ad2_ref[...], ones_col)
              + b2_ref[...].reshape(1, -1))

    g = jnp.dot(jnp.full((1, N), 1.0 / N, jnp.float32), o2)
    out = jax.lax.dot_general(g, fcw_ref[...], _DN)
    out_ref[...] = out.reshape(HID) + fcb_ref[...]


def kernel(roi_feats, W1, att_src1, att_dst1, b1, W2, att_src2, att_dst2, b2,
           fcW, fcb):
    return pl.pallas_call(
        _gat_kernel,
        out_shape=jax.ShapeDtypeStruct((HID,), jnp.float32),
    )(roi_feats, W1, att_src1, att_dst1, b1, W2, att_src2, att_dst2, b2,
      fcW, fcb)
